# baseline (speedup 1.0000x reference)
import functools

import jax
import jax.numpy as jnp
from jax import lax
from jax.experimental import pallas as pl
from jax.experimental.pallas import tpu as pltpu
from jax.experimental.pallas import tpu_sc as plsc

_ROWS_PER_DMA = 128


def _sc_gather(table, idx3d):
    nw, ch, rpd = idx3d.shape
    nchunks = nw * ch
    _, d = table.shape
    info = plsc.get_sparse_core_info()
    nc = info.num_cores

    mesh = plsc.VectorSubcoreMesh(core_axis_name="c", subcore_axis_name="s")

    @functools.partial(
        pl.kernel,
        mesh=mesh,
        out_type=jax.ShapeDtypeStruct((nchunks * rpd, d), jnp.float32),
        scratch_types=[
            pltpu.VMEM((ch, rpd), jnp.int32),
            pltpu.VMEM((rpd, d), jnp.float32),
            pltpu.VMEM((rpd, d), jnp.float32),
            pltpu.SemaphoreType.DMA,
            pltpu.SemaphoreType.DMA,
            pltpu.SemaphoreType.DMA,
            pltpu.SemaphoreType.DMA,
        ],
    )
    def gather_kernel(table_hbm, idx_hbm, e_hbm, idx_v, buf0, buf1,
                      gsem0, gsem1, wsem0, wsem1):
        wid = lax.axis_index("s") * nc + lax.axis_index("c")
        pltpu.sync_copy(idx_hbm.at[wid], idx_v)
        rbase = wid * ch * rpd

        def g(j, buf, sem):
            return pltpu.make_async_copy(table_hbm.at[idx_v.at[j]], buf, sem)

        def w(j, buf, sem):
            return pltpu.make_async_copy(
                buf, e_hbm.at[pl.ds(rbase + j * rpd, rpd)], sem)

        g(0, buf0, gsem0).start()

        def body(jo, carry):
            j0 = 2 * jo
            j1 = j0 + 1

            @pl.when(jo > 0)
            def _():
                w(j0 - 1, buf1, wsem1).wait()

            g(j1, buf1, gsem1).start()
            g(j0, buf0, gsem0).wait()
            w(j0, buf0, wsem0).start()

            @pl.when(jo + 1 < ch // 2)
            def _():
                w(j0, buf0, wsem0).wait()
                g(j0 + 2, buf0, gsem0).start()

            g(j1, buf1, gsem1).wait()
            w(j1, buf1, wsem1).start()
            return carry

        lax.fori_loop(0, ch // 2, body, 0)
        if ch % 2 == 1:
            w(ch - 3, buf0, wsem0).wait()
            g(ch - 1, buf0, gsem0).start()
            w(ch - 2, buf1, wsem1).wait()
            g(ch - 1, buf0, gsem0).wait()
            w(ch - 1, buf0, wsem0).start()
            w(ch - 1, buf0, wsem0).wait()
        else:
            w(ch - 2, buf0, wsem0).wait()
            w(ch - 1, buf1, wsem1).wait()

    return gather_kernel(table, idx3d)


def _tc_body(bb, tt, tok_base, len_ref, e_ref, w_ref, b_ref, out_ref):
    lblk = pl.program_id(1)
    ln = len_ref[...]
    inv = 1.0 / jnp.maximum(ln, 1).astype(jnp.float32)
    wmat = w_ref[...]
    bias = b_ref[...]
    acc = jnp.zeros(out_ref.shape, jnp.float32)
    for t in range(tt):
        tok = tok_base + lblk * tt + t
        e = e_ref[t]
        h = e + jnp.maximum(
            jnp.dot(e, wmat, preferred_element_type=jnp.float32) + bias, 0.0
        )
        m = (tok < ln).astype(jnp.float32)
        acc = acc + h * m
    contrib = acc * inv

    @pl.when(lblk == 0)
    def _():
        out_ref[...] = contrib

    @pl.when(lblk > 0)
    def _():
        out_ref[...] += contrib


def _tc_mean(lengths_col, e3, wmat, brow, bb, tt, tok_base=0):
    ltot, b, d = e3.shape
    grid = (b // bb, ltot // tt)
    return pl.pallas_call(
        functools.partial(_tc_body, bb, tt, tok_base),
        grid=grid,
        in_specs=[
            pl.BlockSpec((bb, 1), lambda i, l: (i, 0)),
            pl.BlockSpec((tt, bb, d), lambda i, l: (l, i, 0)),
            pl.BlockSpec((d, d), lambda i, l: (0, 0)),
            pl.BlockSpec((1, d), lambda i, l: (0, 0)),
        ],
        out_specs=pl.BlockSpec((bb, d), lambda i, l: (i, 0)),
        out_shape=jax.ShapeDtypeStruct((b, d), jnp.float32),
    )(lengths_col, e3, wmat, brow)


_TOKEN_CHUNKS = 5


def kernel(x, initialHidden, lengths, table, W, b):
    del initialHidden
    bsz, seq = x.shape
    _, d = table.shape
    info = plsc.get_sparse_core_info()
    nw = info.num_cores * info.num_subcores
    xt = jnp.where(
        jnp.arange(seq, dtype=jnp.int32)[:, None] < lengths[None, :].astype(jnp.int32),
        x.T.astype(jnp.int32), 0)
    lcol = lengths.astype(jnp.int32).reshape(bsz, 1)
    wt = W.T
    brow = b.reshape(1, d)

    ltok = seq // _TOKEN_CHUNKS
    partials = []
    for c in range(_TOKEN_CHUNKS):
        idx3d = xt[c * ltok:(c + 1) * ltok].reshape(nw, -1, _ROWS_PER_DMA)
        e_flat = _sc_gather(table, idx3d)
        e3 = e_flat.reshape(ltok, bsz, d)
        partials.append(
            _tc_mean(lcol, e3, wt, brow, bb=512, tt=ltok, tok_base=c * ltok))
    out = partials[0]
    for p in partials[1:]:
        out = out + p
    return out


# speedup vs baseline: 27.4332x; 27.4332x over previous
import functools

import jax
import jax.numpy as jnp
from jax import lax
from jax.experimental import pallas as pl
from jax.experimental.pallas import tpu as pltpu
from jax.experimental.pallas import tpu_sc as plsc

_ROWS_PER_DMA = 128


def _sc_gather(table, idx3d):
    nw, ch, rpd = idx3d.shape
    nchunks = nw * ch
    _, d = table.shape
    info = plsc.get_sparse_core_info()
    nc = info.num_cores

    mesh = plsc.VectorSubcoreMesh(core_axis_name="c", subcore_axis_name="s")

    @functools.partial(
        pl.kernel,
        mesh=mesh,
        out_type=jax.ShapeDtypeStruct((nchunks * rpd, d), jnp.float32),
        scratch_types=[
            pltpu.VMEM((ch, rpd), jnp.int32),
            pltpu.VMEM((rpd, d), jnp.float32),
            pltpu.VMEM((rpd, d), jnp.float32),
            pltpu.SemaphoreType.DMA,
            pltpu.SemaphoreType.DMA,
            pltpu.SemaphoreType.DMA,
            pltpu.SemaphoreType.DMA,
        ],
    )
    def gather_kernel(table_hbm, idx_hbm, e_hbm, idx_v, buf0, buf1,
                      gsem0, gsem1, wsem0, wsem1):
        wid = lax.axis_index("s") * nc + lax.axis_index("c")
        pltpu.sync_copy(idx_hbm.at[wid], idx_v)
        rbase = wid * ch * rpd

        def g(j, buf, sem):
            return pltpu.make_async_copy(table_hbm.at[idx_v.at[j]], buf, sem)

        def w(j, buf, sem):
            return pltpu.make_async_copy(
                buf, e_hbm.at[pl.ds(rbase + j * rpd, rpd)], sem)

        g(0, buf0, gsem0).start()

        def body(jo, carry):
            j0 = 2 * jo
            j1 = j0 + 1

            @pl.when(jo > 0)
            def _():
                w(j0 - 1, buf1, wsem1).wait()

            g(j1, buf1, gsem1).start()
            g(j0, buf0, gsem0).wait()
            w(j0, buf0, wsem0).start()

            @pl.when(jo + 1 < ch // 2)
            def _():
                w(j0, buf0, wsem0).wait()
                g(j0 + 2, buf0, gsem0).start()

            g(j1, buf1, gsem1).wait()
            w(j1, buf1, wsem1).start()
            return carry

        lax.fori_loop(0, ch // 2, body, 0)
        if ch % 2 == 1:
            w(ch - 3, buf0, wsem0).wait()
            g(ch - 1, buf0, gsem0).start()
            w(ch - 2, buf1, wsem1).wait()
            g(ch - 1, buf0, gsem0).wait()
            w(ch - 1, buf0, wsem0).start()
            w(ch - 1, buf0, wsem0).wait()
        else:
            w(ch - 2, buf0, wsem0).wait()
            w(ch - 1, buf1, wsem1).wait()

    return gather_kernel(table, idx3d)


def _tc_body(bb, tt, tok_base, len_ref, e_ref, w_ref, b_ref, out_ref):
    lblk = pl.program_id(1)
    ln = len_ref[...]
    inv = 1.0 / jnp.maximum(ln, 1).astype(jnp.float32)
    wmat = w_ref[...]
    bias = b_ref[...]
    acc = jnp.zeros(out_ref.shape, jnp.float32)
    for t in range(tt):
        tok = tok_base + lblk * tt + t
        e = e_ref[t]
        h = e + jnp.maximum(
            jnp.dot(e, wmat, preferred_element_type=jnp.float32) + bias, 0.0
        )
        m = (tok < ln).astype(jnp.float32)
        acc = acc + h * m
    contrib = acc * inv

    @pl.when(lblk == 0)
    def _():
        out_ref[...] = contrib

    @pl.when(lblk > 0)
    def _():
        out_ref[...] += contrib


def _tc_mean(lengths_col, e3, wmat, brow, bb, tt, tok_base=0):
    ltot, b, d = e3.shape
    grid = (b // bb, ltot // tt)
    return pl.pallas_call(
        functools.partial(_tc_body, bb, tt, tok_base),
        grid=grid,
        in_specs=[
            pl.BlockSpec((bb, 1), lambda i, l: (i, 0)),
            pl.BlockSpec((tt, bb, d), lambda i, l: (l, i, 0)),
            pl.BlockSpec((d, d), lambda i, l: (0, 0)),
            pl.BlockSpec((1, d), lambda i, l: (0, 0)),
        ],
        out_specs=pl.BlockSpec((bb, d), lambda i, l: (i, 0)),
        out_shape=jax.ShapeDtypeStruct((b, d), jnp.float32),
    )(lengths_col, e3, wmat, brow)


_TOKEN_CHUNKS = 5


def kernel(x, initialHidden, lengths, table, W, b):
    del initialHidden
    bsz, seq = x.shape
    _, d = table.shape
    info = plsc.get_sparse_core_info()
    nw = info.num_cores * info.num_subcores
    xt = x.T.astype(jnp.int32)
    lcol = lengths.astype(jnp.int32).reshape(bsz, 1)
    wt = W.T
    brow = b.reshape(1, d)

    ltok = seq // _TOKEN_CHUNKS
    partials = []
    for c in range(_TOKEN_CHUNKS):
        idx3d = xt[c * ltok:(c + 1) * ltok].reshape(nw, -1, _ROWS_PER_DMA)
        e_flat = _sc_gather(table, idx3d)
        e3 = e_flat.reshape(ltok, bsz, d)
        partials.append(
            _tc_mean(lcol, e3, wt, brow, bb=1024, tt=ltok, tok_base=c * ltok))
    out = partials[0]
    for p in partials[1:]:
        out = out + p
    return out


# speedup vs baseline: 28.2503x; 1.0298x over previous
import functools

import jax
import jax.numpy as jnp
from jax import lax
from jax.experimental import pallas as pl
from jax.experimental.pallas import tpu as pltpu
from jax.experimental.pallas import tpu_sc as plsc

_ROWS_PER_DMA = 128


def _sc_gather(table, idx3d):
    nw, ch, rpd = idx3d.shape
    nchunks = nw * ch
    _, d = table.shape
    info = plsc.get_sparse_core_info()
    nc = info.num_cores

    mesh = plsc.VectorSubcoreMesh(core_axis_name="c", subcore_axis_name="s")

    nbuf = min(4, ch)
    assert ch <= 20, "keep the static unroll well under the TileTask size cap"

    @functools.partial(
        pl.kernel,
        mesh=mesh,
        out_type=jax.ShapeDtypeStruct((nchunks * rpd, d), jnp.float32),
        scratch_types=[pltpu.VMEM((ch, rpd), jnp.int32)]
        + [pltpu.VMEM((rpd, d), jnp.float32) for _ in range(nbuf)]
        + [pltpu.SemaphoreType.DMA for _ in range(2 * nbuf)],
    )
    def gather_kernel(table_hbm, idx_hbm, e_hbm, idx_v, *scratch):
        bufs = scratch[:nbuf]
        gsems = scratch[nbuf:2 * nbuf]
        wsems = scratch[2 * nbuf:]
        wid = lax.axis_index("s") * nc + lax.axis_index("c")
        pltpu.sync_copy(idx_hbm.at[wid], idx_v)
        rbase = wid * ch * rpd

        def g(j, k):
            return pltpu.make_async_copy(table_hbm.at[idx_v.at[j]], bufs[k],
                                         gsems[k])

        def w(j, k):
            return pltpu.make_async_copy(
                bufs[k], e_hbm.at[pl.ds(rbase + j * rpd, rpd)], wsems[k])

        for j in range(nbuf):
            g(j, j).start()
        for j in range(ch):
            k = j % nbuf
            g(j, k).wait()
            w(j, k).start()
            if j + nbuf < ch:
                w(j, k).wait()
                g(j + nbuf, k).start()
        for j in range(max(0, ch - nbuf), ch):
            w(j, j % nbuf).wait()

    return gather_kernel(table, idx3d)


def _tc_body(bb, tt, tok_base, len_ref, e_ref, w_ref, b_ref, out_ref):
    lblk = pl.program_id(1)
    ln = len_ref[...]
    inv = 1.0 / jnp.maximum(ln, 1).astype(jnp.float32)
    wmat = w_ref[...]
    bias = b_ref[...]
    acc = jnp.zeros(out_ref.shape, jnp.float32)
    for t in range(tt):
        tok = tok_base + lblk * tt + t
        e = e_ref[t]
        h = e + jnp.maximum(
            jnp.dot(e, wmat, preferred_element_type=jnp.float32) + bias, 0.0
        )
        m = (tok < ln).astype(jnp.float32)
        acc = acc + h * m
    contrib = acc * inv

    @pl.when(lblk == 0)
    def _():
        out_ref[...] = contrib

    @pl.when(lblk > 0)
    def _():
        out_ref[...] += contrib


def _tc_mean(lengths_col, e3, wmat, brow, bb, tt, tok_base=0):
    ltot, b, d = e3.shape
    grid = (b // bb, ltot // tt)
    return pl.pallas_call(
        functools.partial(_tc_body, bb, tt, tok_base),
        grid=grid,
        in_specs=[
            pl.BlockSpec((bb, 1), lambda i, l: (i, 0)),
            pl.BlockSpec((tt, bb, d), lambda i, l: (l, i, 0)),
            pl.BlockSpec((d, d), lambda i, l: (0, 0)),
            pl.BlockSpec((1, d), lambda i, l: (0, 0)),
        ],
        out_specs=pl.BlockSpec((bb, d), lambda i, l: (i, 0)),
        out_shape=jax.ShapeDtypeStruct((b, d), jnp.float32),
    )(lengths_col, e3, wmat, brow)


_TOKEN_CHUNKS = 5


def kernel(x, initialHidden, lengths, table, W, b):
    del initialHidden
    bsz, seq = x.shape
    _, d = table.shape
    info = plsc.get_sparse_core_info()
    nw = info.num_cores * info.num_subcores
    xt = x.T.astype(jnp.int32)
    lcol = lengths.astype(jnp.int32).reshape(bsz, 1)
    wt = W.T
    brow = b.reshape(1, d)

    ltok = seq // _TOKEN_CHUNKS
    partials = []
    for c in range(_TOKEN_CHUNKS):
        idx3d = xt[c * ltok:(c + 1) * ltok].reshape(nw, -1, _ROWS_PER_DMA)
        e_flat = _sc_gather(table, idx3d)
        e3 = e_flat.reshape(ltok, bsz, d)
        partials.append(
            _tc_mean(lcol, e3, wt, brow, bb=1024, tt=ltok, tok_base=c * ltok))
    out = partials[0]
    for p in partials[1:]:
        out = out + p
    return out
